# baseline (speedup 1.0000x reference)
import functools

import jax
import jax.numpy as jnp
from jax import lax
from jax.experimental import pallas as pl
from jax.experimental.pallas import tpu as pltpu
from jax.experimental.pallas import tpu_sc as plsc

N = 10000
E = 320000
D = 128
DP = 136
NC = 2
NS = 16
L = 16
NW = NC * NS
PER_TILE = E // NW
C = 80
CHUNKS = PER_TILE // C
ROWS_PER_TILE = N // NS
EPR = D // L


def _prep_body(h_ref, w_ref, wb_ref, aw_ref, wh_ref, sd_ref, ss_ref):
    hp = jax.lax.Precision.HIGHEST
    wh = lax.dot_general(h_ref[...], w_ref[...],
                         (((1,), (1,)), ((), ())), precision=hp)
    wh = wh + wb_ref[...].reshape(1, D)
    wh_ref[...] = wh
    a1 = aw_ref[:, 0:D]
    a2 = aw_ref[:, D:2 * D]
    sd_ref[...] = jnp.sum(wh * a1, axis=1)
    ss_ref[...] = jnp.sum(wh * a2, axis=1)


_prep = pl.pallas_call(
    _prep_body,
    out_shape=(
        jax.ShapeDtypeStruct((N, D), jnp.float32),
        jax.ShapeDtypeStruct((N,), jnp.float32),
        jax.ShapeDtypeStruct((N,), jnp.float32),
    ),
)

EH1 = 163840
EH2 = E - EH1


def _make_se(e_h):
    rows = e_h // EPR
    blk = rows // 5

    def _se_body(ea_ref, aw_ref, ab_ref, se_ref):
        ea = ea_ref[...].reshape(blk, D)
        a3 = aw_ref[:, 2 * D:]
        a3t = jnp.concatenate([a3] * EPR, axis=1)
        ii = lax.broadcasted_iota(jnp.int32, (D, D), 0)
        jj = lax.broadcasted_iota(jnp.int32, (D, D), 1)
        seg = jnp.where(ii // 16 == jj, 1.0, 0.0)
        se = lax.dot_general(ea * a3t, seg, (((1,), (0,)), ((), ())),
                             precision=jax.lax.Precision.HIGHEST)
        se_ref[...] = se + ab_ref[...].reshape(1, 1)

    return pl.pallas_call(
        _se_body,
        grid=(rows // blk,),
        in_specs=[
            pl.BlockSpec((blk * D,), lambda i: (i,)),
            pl.BlockSpec((1, 272), lambda i: (0, 0)),
            pl.BlockSpec((1,), lambda i: (0,)),
        ],
        out_specs=pl.BlockSpec((blk, D), lambda i: (i, 0)),
        out_shape=jax.ShapeDtypeStruct((rows, D), jnp.float32),
    )


_se_prep1 = _make_se(EH1)
_se_prep2 = _make_se(EH2)


_mesh = plsc.VectorSubcoreMesh(core_axis_name="c", subcore_axis_name="s")

_BUF = dict(
    rowb=pltpu.VMEM((C,), jnp.int32),
    colb=pltpu.VMEM((C,), jnp.int32),
    seb=pltpu.VMEM((C // EPR * D,), jnp.float32),
    alphab=pltpu.VMEM((C,), jnp.float32),
    rowscat=pltpu.VMEM((C,), jnp.int32),
    normb=pltpu.VMEM((C, DP - D), jnp.float32),
    gbuf=pltpu.VMEM((C, DP), jnp.float32),
    rsem=pltpu.SemaphoreType.DMA,
    csem=pltpu.SemaphoreType.DMA,
    esem=pltpu.SemaphoreType.DMA,
    gsem=pltpu.SemaphoreType.DMA,
    ssem=pltpu.SemaphoreType.DMA,
)


def _make_sc(e_h, eoff):
  per_tile = e_h // NW
  n_chunks = per_tile // C

  @functools.partial(
      pl.kernel,
      out_type=(jax.ShapeDtypeStruct((NC, N, D), jnp.float32),
                jax.ShapeDtypeStruct((NC, N, DP - D), jnp.float32)),
      mesh=_mesh,
      compiler_params=pltpu.CompilerParams(use_tc_tiling_on_sc=False,
                                           needs_layout_passes=False),
      scratch_types=[
          pltpu.VMEM((N,), jnp.float32),
          pltpu.VMEM((N,), jnp.float32),
          pltpu.VMEM_SHARED((N, DP), jnp.float32),
      ] + list(_BUF.values()) * 2,
  )
  def _sc_edges(whp_hbm, sd_hbm, ss_hbm, se_hbm, ei_hbm,
                feat_hbm, norm_hbm, sdv, ssv, acc, *bufs):
    CHUNKS = n_chunks
    PER_TILE = per_tile
    nb = len(_BUF)
    A = dict(zip(_BUF.keys(), bufs[:nb]))
    B = dict(zip(_BUF.keys(), bufs[nb:]))

    c = lax.axis_index("c")
    s = lax.axis_index("s")
    wid = c * NS + s
    base = wid * PER_TILE
    nbase = s * ROWS_PER_TILE

    lane = lax.iota(jnp.int32, L)
    se_pat = (lane // EPR) * D + (lane % EPR)

    zeros = jnp.zeros((L,), jnp.float32)
    gz = A["gbuf"]

    @pl.loop(0, C)
    def _(j):
        for q in range(D // L):
            gz[j, pl.ds(q * L, L)] = zeros
        gz[j, pl.ds(DP - L, L)] = zeros

    @pl.loop(0, ROWS_PER_TILE // C)
    def _(i):
        pltpu.sync_copy(gz, acc.at[pl.ds(nbase + i * C, C)])
    rem = ROWS_PER_TILE % C
    if rem:
        pltpu.sync_copy(gz.at[pl.ds(0, rem)],
                        acc.at[pl.ds(nbase + (ROWS_PER_TILE // C) * C, rem)])

    pltpu.sync_copy(sd_hbm, sdv)
    pltpu.sync_copy(ss_hbm, ssv)

    plsc.subcore_barrier()

    SEW = C // EPR * D

    def issue_idx(k, buf):
        off = base + k * C
        goff = eoff + off
        pltpu.async_copy(ei_hbm.at[0, pl.ds(goff, C)], buf["rowb"],
                         buf["rsem"])
        pltpu.async_copy(ei_hbm.at[1, pl.ds(goff, C)], buf["colb"],
                         buf["csem"])
        pltpu.async_copy(se_hbm.at[pl.ds(off // EPR * D, SEW)], buf["seb"],
                         buf["esem"])

    def wait_idx(k, buf):
        off = base + k * C
        goff = eoff + off
        pltpu.make_async_copy(ei_hbm.at[0, pl.ds(goff, C)], buf["rowb"],
                              buf["rsem"]).wait()
        pltpu.make_async_copy(ei_hbm.at[1, pl.ds(goff, C)], buf["colb"],
                              buf["csem"]).wait()
        pltpu.make_async_copy(se_hbm.at[pl.ds(off // EPR * D, SEW)],
                              buf["seb"], buf["esem"]).wait()

    def issue_gather(buf):
        pltpu.async_copy(whp_hbm.at[buf["colb"]], buf["gbuf"], buf["gsem"])

    def wait_gather(buf):
        pltpu.make_async_copy(whp_hbm.at[buf["colb"]], buf["gbuf"],
                              buf["gsem"]).wait()

    def issue_scatter(buf):
        pltpu.async_copy(buf["gbuf"], acc.at[buf["rowscat"]], buf["ssem"],
                         add=True)

    def wait_scatter(buf):
        pltpu.make_async_copy(buf["gbuf"], acc.at[buf["rowscat"]],
                              buf["ssem"]).wait()

    def compute_alpha(buf):
        for g in range(C // L):
            rv = buf["rowb"][pl.ds(g * L, L)]
            cv = buf["colb"][pl.ds(g * L, L)]
            se = plsc.load_gather(buf["seb"], [se_pat + g * (2 * D)])
            x = plsc.load_gather(sdv, [rv]) + plsc.load_gather(ssv, [cv]) + se
            x = jnp.where(x > 0, x, x * 0.01)
            buf["alphab"][pl.ds(g * L, L)] = jnp.exp(x)

    def scale(buf):
        gb = buf["gbuf"]

        @plsc.parallel_loop(0, C, unroll=4)
        def _(j):
            ab = plsc.load_gather(buf["alphab"],
                                  [jnp.zeros((L,), jnp.int32) + j])
            for q in range(D // L):
                gb[j, pl.ds(q * L, L)] = gb[j, pl.ds(q * L, L)] * ab
            v = gb[j, pl.ds(DP - L, L)]
            gb[j, pl.ds(DP - L, L)] = jnp.where(lane < D - (DP - L),
                                                v, v * ab)

    def save_rowscat(buf):
        for g in range(C // L):
            buf["rowscat"][pl.ds(g * L, L)] = buf["rowb"][pl.ds(g * L, L)]

    issue_idx(0, A)
    wait_idx(0, A)
    issue_gather(A)

    def body(k, cur, nxt, first=False):
        issue_idx(k + 1, nxt)
        compute_alpha(cur)
        wait_gather(cur)
        if not first:
            wait_scatter(nxt)
        wait_idx(k + 1, nxt)
        issue_gather(nxt)
        scale(cur)
        save_rowscat(cur)
        issue_scatter(cur)

    body(0, A, B, first=True)

    m = (CHUNKS - 2) // 2

    @pl.loop(0, m)
    def _(t):
        k = 1 + 2 * t
        body(k, B, A)
        body(k + 1, A, B)

    for k in range(2 * m + 1, CHUNKS - 1):
        body(k, A if k % 2 == 0 else B, B if k % 2 == 0 else A)
    fc, fo = (A, B) if (CHUNKS - 1) % 2 == 0 else (B, A)
    compute_alpha(fc)
    wait_gather(fc)
    wait_scatter(fo)
    scale(fc)
    save_rowscat(fc)
    issue_scatter(fc)
    wait_scatter(fc)

    plsc.subcore_barrier()

    pltpu.sync_copy(acc.at[pl.ds(nbase, ROWS_PER_TILE), pl.ds(0, D)],
                    feat_hbm.at[c].at[pl.ds(nbase, ROWS_PER_TILE)])

    @pl.loop(0, ROWS_PER_TILE // C)
    def _(i):
        r = nbase + i * C
        pltpu.sync_copy(acc.at[pl.ds(r, C), pl.ds(D, DP - D)], A["normb"])
        pltpu.sync_copy(A["normb"], norm_hbm.at[c].at[pl.ds(r, C)])
    remn = ROWS_PER_TILE % C
    if remn:
        r = nbase + (ROWS_PER_TILE // C) * C
        pltpu.sync_copy(acc.at[pl.ds(r, remn), pl.ds(D, DP - D)],
                        A["normb"].at[pl.ds(0, remn)])
        pltpu.sync_copy(A["normb"].at[pl.ds(0, remn)],
                        norm_hbm.at[c].at[pl.ds(r, remn)])

  return _sc_edges


_sc_edges1 = _make_sc(EH1, 0)
_sc_edges2 = _make_sc(EH2, EH1)


def _fin_body(f1_ref, n1_ref, f2_ref, n2_ref, o_ref):
    fsum = f1_ref[0] + f1_ref[1] + f2_ref[0] + f2_ref[1]
    nsum = (n1_ref[0, :, 0:1] + n1_ref[1, :, 0:1]
            + n2_ref[0, :, 0:1] + n2_ref[1, :, 0:1])
    o_ref[...] = fsum / (nsum + 1e-8)


_fin = pl.pallas_call(
    _fin_body,
    out_shape=jax.ShapeDtypeStruct((N, D), jnp.float32),
)


def kernel(h, edge_index, edge_attr, w_weight, w_bias, a_weight, a_bias):
    wh, sd, ss = _prep(h, w_weight, w_bias, a_weight)
    pad = jnp.concatenate(
        [jnp.ones((N, 1), jnp.float32), jnp.zeros((N, DP - D - 1), jnp.float32)],
        axis=1)
    whp = jnp.concatenate([wh, pad], axis=1)
    se1 = _se_prep1(edge_attr[:EH1].reshape(EH1 * 16), a_weight, a_bias)
    f1, n1 = _sc_edges1(whp, sd, ss, se1.reshape(EH1 // EPR * D), edge_index)
    se2 = _se_prep2(edge_attr[EH1:].reshape(EH2 * 16), a_weight, a_bias)
    f2, n2 = _sc_edges2(whp, sd, ss, se2.reshape(EH2 // EPR * D), edge_index)
    return _fin(f1, n1, f2, n2)


# speedup vs baseline: 1.0128x; 1.0128x over previous
import functools

import jax
import jax.numpy as jnp
from jax import lax
from jax.experimental import pallas as pl
from jax.experimental.pallas import tpu as pltpu
from jax.experimental.pallas import tpu_sc as plsc

N = 10000
E = 320000
D = 128
DP = 136
NC = 2
NS = 16
L = 16
NW = NC * NS
PER_TILE = E // NW
C = 80
CHUNKS = PER_TILE // C
ROWS_PER_TILE = N // NS
EPR = D // L


def _prep_body(h_ref, w_ref, wb_ref, aw_ref, wh_ref, sd_ref, ss_ref):
    hp = jax.lax.Precision.HIGHEST
    wh = lax.dot_general(h_ref[...], w_ref[...],
                         (((1,), (1,)), ((), ())), precision=hp)
    wh = wh + wb_ref[...].reshape(1, D)
    wh_ref[...] = wh
    a1 = aw_ref[:, 0:D]
    a2 = aw_ref[:, D:2 * D]
    sd_ref[...] = jnp.sum(wh * a1, axis=1)
    ss_ref[...] = jnp.sum(wh * a2, axis=1)


_prep = pl.pallas_call(
    _prep_body,
    out_shape=(
        jax.ShapeDtypeStruct((N, D), jnp.float32),
        jax.ShapeDtypeStruct((N,), jnp.float32),
        jax.ShapeDtypeStruct((N,), jnp.float32),
    ),
)

_SE_ROWS = E // EPR
_SE_BLK = 4000


def _se_body(ea_ref, aw_ref, ab_ref, se_ref):
    ea = ea_ref[...].reshape(_SE_BLK, D)
    a3 = aw_ref[:, 2 * D:]
    a3t = jnp.concatenate([a3] * EPR, axis=1)
    ii = lax.broadcasted_iota(jnp.int32, (D, D), 0)
    jj = lax.broadcasted_iota(jnp.int32, (D, D), 1)
    seg = jnp.where(ii // 16 == jj, 1.0, 0.0)
    se = lax.dot_general(ea * a3t, seg, (((1,), (0,)), ((), ())),
                         precision=jax.lax.Precision.HIGHEST)
    se_ref[...] = se + ab_ref[...].reshape(1, 1)


_se_prep = pl.pallas_call(
    _se_body,
    grid=(_SE_ROWS // _SE_BLK,),
    in_specs=[
        pl.BlockSpec((_SE_BLK * D,), lambda i: (i,)),
        pl.BlockSpec((1, 272), lambda i: (0, 0)),
        pl.BlockSpec((1,), lambda i: (0,)),
    ],
    out_specs=pl.BlockSpec((_SE_BLK, D), lambda i: (i, 0)),
    out_shape=jax.ShapeDtypeStruct((_SE_ROWS, D), jnp.float32),
)


_mesh = plsc.VectorSubcoreMesh(core_axis_name="c", subcore_axis_name="s")

_BUF = dict(
    rowb=pltpu.VMEM((C,), jnp.int32),
    colb=pltpu.VMEM((C,), jnp.int32),
    seb=pltpu.VMEM((C // EPR * D,), jnp.float32),
    alphab=pltpu.VMEM((C,), jnp.float32),
    rowscat=pltpu.VMEM((C,), jnp.int32),
    normb=pltpu.VMEM((C, DP - D), jnp.float32),
    gbuf=pltpu.VMEM((C, DP), jnp.float32),
    rsem=pltpu.SemaphoreType.DMA,
    csem=pltpu.SemaphoreType.DMA,
    esem=pltpu.SemaphoreType.DMA,
    gsem=pltpu.SemaphoreType.DMA,
    ssem=pltpu.SemaphoreType.DMA,
)


@functools.partial(
    pl.kernel,
    out_type=(jax.ShapeDtypeStruct((NC, N, D), jnp.float32),
              jax.ShapeDtypeStruct((NC, N, DP - D), jnp.float32)),
    mesh=_mesh,
    compiler_params=pltpu.CompilerParams(use_tc_tiling_on_sc=False,
                                         needs_layout_passes=False),
    scratch_types=[
        pltpu.VMEM((N,), jnp.float32),
        pltpu.VMEM((N,), jnp.float32),
        pltpu.VMEM_SHARED((N, DP), jnp.float32),
    ] + list(_BUF.values()) * 2,
)
def _sc_edges(whp_hbm, sd_hbm, ss_hbm, se_hbm, ei_hbm,
              feat_hbm, norm_hbm, sdv, ssv, acc, *bufs):
    nb = len(_BUF)
    A = dict(zip(_BUF.keys(), bufs[:nb]))
    B = dict(zip(_BUF.keys(), bufs[nb:]))

    c = lax.axis_index("c")
    s = lax.axis_index("s")
    wid = c * NS + s
    base = wid * PER_TILE
    nbase = s * ROWS_PER_TILE

    lane = lax.iota(jnp.int32, L)
    se_pat = (lane // EPR) * D + (lane % EPR)

    zeros = jnp.zeros((L,), jnp.float32)
    gz = A["gbuf"]

    @pl.loop(0, C)
    def _(j):
        for q in range(D // L):
            gz[j, pl.ds(q * L, L)] = zeros
        gz[j, pl.ds(DP - L, L)] = zeros

    @pl.loop(0, ROWS_PER_TILE // C)
    def _(i):
        pltpu.sync_copy(gz, acc.at[pl.ds(nbase + i * C, C)])
    rem = ROWS_PER_TILE % C
    if rem:
        pltpu.sync_copy(gz.at[pl.ds(0, rem)],
                        acc.at[pl.ds(nbase + (ROWS_PER_TILE // C) * C, rem)])

    pltpu.sync_copy(sd_hbm, sdv)
    pltpu.sync_copy(ss_hbm, ssv)

    plsc.subcore_barrier()

    SEW = C // EPR * D

    def issue_idx(k, buf):
        off = base + k * C
        pltpu.async_copy(ei_hbm.at[0, pl.ds(off, C)], buf["rowb"],
                         buf["rsem"])
        pltpu.async_copy(ei_hbm.at[1, pl.ds(off, C)], buf["colb"],
                         buf["csem"])
        pltpu.async_copy(se_hbm.at[pl.ds(off // EPR * D, SEW)], buf["seb"],
                         buf["esem"])

    def wait_idx(k, buf):
        off = base + k * C
        pltpu.make_async_copy(ei_hbm.at[0, pl.ds(off, C)], buf["rowb"],
                              buf["rsem"]).wait()
        pltpu.make_async_copy(ei_hbm.at[1, pl.ds(off, C)], buf["colb"],
                              buf["csem"]).wait()
        pltpu.make_async_copy(se_hbm.at[pl.ds(off // EPR * D, SEW)],
                              buf["seb"], buf["esem"]).wait()

    def issue_gather(buf):
        pltpu.async_copy(whp_hbm.at[buf["colb"]], buf["gbuf"], buf["gsem"])

    def wait_gather(buf):
        pltpu.make_async_copy(whp_hbm.at[buf["colb"]], buf["gbuf"],
                              buf["gsem"]).wait()

    def issue_scatter(buf):
        pltpu.async_copy(buf["gbuf"], acc.at[buf["rowscat"]], buf["ssem"],
                         add=True)

    def wait_scatter(buf):
        pltpu.make_async_copy(buf["gbuf"], acc.at[buf["rowscat"]],
                              buf["ssem"]).wait()

    def compute_alpha(buf):
        for g in range(C // L):
            rv = buf["rowb"][pl.ds(g * L, L)]
            cv = buf["colb"][pl.ds(g * L, L)]
            se = plsc.load_gather(buf["seb"], [se_pat + g * (2 * D)])
            x = plsc.load_gather(sdv, [rv]) + plsc.load_gather(ssv, [cv]) + se
            x = jnp.where(x > 0, x, x * 0.01)
            buf["alphab"][pl.ds(g * L, L)] = jnp.exp(x)

    def scale(buf):
        gb = buf["gbuf"]

        @plsc.parallel_loop(0, C, unroll=4)
        def _(j):
            ab = plsc.load_gather(buf["alphab"],
                                  [jnp.zeros((L,), jnp.int32) + j])
            for q in range(D // L):
                gb[j, pl.ds(q * L, L)] = gb[j, pl.ds(q * L, L)] * ab
            v = gb[j, pl.ds(DP - L, L)]
            gb[j, pl.ds(DP - L, L)] = jnp.where(lane < D - (DP - L),
                                                v, v * ab)

    def save_rowscat(buf):
        for g in range(C // L):
            buf["rowscat"][pl.ds(g * L, L)] = buf["rowb"][pl.ds(g * L, L)]

    issue_idx(0, A)
    wait_idx(0, A)
    issue_gather(A)

    def body(k, cur, nxt, first=False):
        issue_idx(k + 1, nxt)
        compute_alpha(cur)
        wait_gather(cur)
        if not first:
            wait_scatter(nxt)
        wait_idx(k + 1, nxt)
        issue_gather(nxt)
        scale(cur)
        save_rowscat(cur)
        issue_scatter(cur)

    body(0, A, B, first=True)

    @pl.loop(0, (CHUNKS - 3) // 2)
    def _(t):
        k = 1 + 2 * t
        body(k, B, A)
        body(k + 1, A, B)

    body(CHUNKS - 2, B, A)
    compute_alpha(A)
    wait_gather(A)
    wait_scatter(B)
    scale(A)
    save_rowscat(A)
    issue_scatter(A)
    wait_scatter(A)

    plsc.subcore_barrier()

    pltpu.sync_copy(acc.at[pl.ds(nbase, ROWS_PER_TILE), pl.ds(0, D)],
                    feat_hbm.at[c].at[pl.ds(nbase, ROWS_PER_TILE)])

    @pl.loop(0, ROWS_PER_TILE // C)
    def _(i):
        r = nbase + i * C
        pltpu.sync_copy(acc.at[pl.ds(r, C), pl.ds(D, DP - D)], A["normb"])
        pltpu.sync_copy(A["normb"], norm_hbm.at[c].at[pl.ds(r, C)])
    remn = ROWS_PER_TILE % C
    if remn:
        r = nbase + (ROWS_PER_TILE // C) * C
        pltpu.sync_copy(acc.at[pl.ds(r, remn), pl.ds(D, DP - D)],
                        A["normb"].at[pl.ds(0, remn)])
        pltpu.sync_copy(A["normb"].at[pl.ds(0, remn)],
                        norm_hbm.at[c].at[pl.ds(r, remn)])


def _fin_body(f_ref, n_ref, o_ref):
    fsum = f_ref[0] + f_ref[1]
    nsum = n_ref[0, :, 0:1] + n_ref[1, :, 0:1]
    o_ref[...] = fsum / (nsum + 1e-8)


_fin = pl.pallas_call(
    _fin_body,
    out_shape=jax.ShapeDtypeStruct((N, D), jnp.float32),
)


def kernel(h, edge_index, edge_attr, w_weight, w_bias, a_weight, a_bias):
    wh, sd, ss = _prep(h, w_weight, w_bias, a_weight)
    pad = jnp.concatenate(
        [jnp.ones((N, 1), jnp.float32), jnp.zeros((N, DP - D - 1), jnp.float32)],
        axis=1)
    whp = jnp.concatenate([wh, pad], axis=1)
    se = _se_prep(edge_attr.reshape(E * 16), a_weight, a_bias)
    feat, norm = _sc_edges(whp, sd, ss, se.reshape(_SE_ROWS * D), edge_index)
    return _fin(feat, norm)
